# Initial kernel scaffold; baseline (speedup 1.0000x reference)
import jax
import jax.numpy as jnp
from jax.experimental import pallas as pl


def kernel(instruction, prop_embeds, node_attrs, edge_attrs, node_graph_ids, edge_indices, Ws_property, W_state, W_relation):
    raise NotImplementedError("write your pallas kernel here")


# speedup vs baseline: 5.7580x; 5.7580x over previous
import functools

import jax
import jax.numpy as jnp
from jax import lax
from jax.experimental import pallas as pl
from jax.experimental.pallas import tpu as pltpu
from jax.experimental.pallas import tpu_sc as plsc

G = 64
H = 256
P = 16
N = 10000
E = 160000

NPAD = 10240
EPAD = 163840
NTILES = 32
CH = EPAD // NTILES
KROWS = CH // 128
NROWS = NPAD // 128

BN = 400
BE = 8000


def _softmax_cols(pe, instr):
    pb = pe.astype(jnp.bfloat16).astype(jnp.float32)
    ib = instr.astype(jnp.bfloat16).astype(jnp.float32)
    logits = jnp.sum(pb * ib, axis=1, keepdims=True)
    m = jnp.max(logits, axis=0, keepdims=True)
    ex = jnp.exp(logits - m)
    return ex / jnp.sum(ex, axis=0, keepdims=True)


def _elu(x):
    return jnp.where(x > 0, x, jnp.exp(jnp.minimum(x, 0.0)) - 1.0)


def _node_body(instr_ref, pe_ref, x_ref, w_ref, wstate_ref, out_ref):
    c = _softmax_cols(pe_ref[...], instr_ref[...])
    acc = jnp.zeros((BN, H), jnp.float32)
    for p in range(P - 1):
        cp = lax.slice(c, (p, 0), (p + 1, 1))
        acc = acc + jnp.dot(x_ref[:, p, :], w_ref[p],
                            preferred_element_type=jnp.float32) * cp
    y = _elu(acc * instr_ref[...])
    out_ref[...] = jnp.dot(y, wstate_ref[...],
                           preferred_element_type=jnp.float32)


def _node_state_vals(instr2d, prop_embeds, node_attrs, w_node, wstate_col):
    return pl.pallas_call(
        _node_body,
        grid=(N // BN,),
        in_specs=[
            pl.BlockSpec((1, H), lambda i: (0, 0)),
            pl.BlockSpec((P, H), lambda i: (0, 0)),
            pl.BlockSpec((BN, P - 1, H), lambda i: (i, 0, 0)),
            pl.BlockSpec((P - 1, H, H), lambda i: (0, 0, 0)),
            pl.BlockSpec((H, 1), lambda i: (0, 0)),
        ],
        out_specs=pl.BlockSpec((BN, 1), lambda i: (i, 0)),
        out_shape=jax.ShapeDtypeStruct((N, 1), jnp.float32),
    )(instr2d, prop_embeds, node_attrs, w_node, wstate_col)


def _edge_body(instr_ref, we_ref, x_ref, wrel_ref, out_ref):
    pre = jnp.dot(x_ref[...], we_ref[...],
                  preferred_element_type=jnp.float32) * instr_ref[...]
    y = _elu(pre)
    out_ref[...] = jnp.dot(y, wrel_ref[...],
                           preferred_element_type=jnp.float32)


def _edge_scalars(instr2d, w_edge, edge_attrs, wrel_col):
    return pl.pallas_call(
        _edge_body,
        grid=(E // BE,),
        in_specs=[
            pl.BlockSpec((1, H), lambda i: (0, 0)),
            pl.BlockSpec((H, H), lambda i: (0, 0)),
            pl.BlockSpec((BE, H), lambda i: (i, 0)),
            pl.BlockSpec((H, 1), lambda i: (0, 0)),
        ],
        out_specs=pl.BlockSpec((BE, 1), lambda i: (i, 0)),
        out_shape=jax.ShapeDtypeStruct((E, 1), jnp.float32),
    )(instr2d, w_edge, edge_attrs, wrel_col)


def _dist_body(gids_ref, out_ref):
    gids = gids_ref[...]
    g3 = lax.broadcasted_iota(jnp.int32, (G, NROWS, 128), 0)
    oh = (gids[None, :, :] == g3).astype(jnp.float32)
    cnt = jnp.sum(oh, axis=(1, 2), keepdims=True)
    inv = 1.0 / jnp.maximum(cnt, 1.0)
    out_ref[...] = jnp.sum(oh * inv, axis=0)


def _distribution(gids2d):
    return pl.pallas_call(
        _dist_body,
        out_shape=jax.ShapeDtypeStruct((NROWS, 128), jnp.float32),
    )(gids2d)


def _make_sc_scatter():
    mesh = plsc.VectorSubcoreMesh(core_axis_name="c", subcore_axis_name="s",
                                  num_cores=2, num_subcores=16)

    @functools.partial(
        pl.kernel,
        mesh=mesh,
        out_type=jax.ShapeDtypeStruct((2, NPAD), jnp.float32),
        scratch_types=[
            pltpu.VMEM((CH,), jnp.float32),
            pltpu.VMEM((KROWS, 128), jnp.int32),
            pltpu.VMEM((KROWS, 128), jnp.int32),
            pltpu.VMEM((CH,), jnp.float32),
            pltpu.VMEM((CH,), jnp.float32),
            pltpu.VMEM_SHARED((NPAD,), jnp.float32),
            pltpu.SemaphoreType.DMA,
        ],
    )
    def sc_scatter(dist_hbm, src_hbm, dst_hbm, s_hbm, out_hbm,
                   d_v, src_v, dst_v, s_v, w_v, acc_sh, sem):
        cid = lax.axis_index("c")
        sid = lax.axis_index("s")
        wid = cid * 16 + sid

        @pl.when(sid == 0)
        def _():
            def zb(i, carry):
                w_v[pl.ds(i * 16, 16)] = jnp.zeros((16,), jnp.float32)
                return carry
            lax.fori_loop(0, CH // 16, zb, 0)
            for r in range(NPAD // CH):
                pltpu.sync_copy(w_v, acc_sh.at[pl.ds(r * CH, CH)])

        plsc.subcore_barrier()

        pltpu.sync_copy(src_hbm.at[wid], src_v)
        pltpu.sync_copy(dst_hbm.at[wid], dst_v)
        pltpu.sync_copy(s_hbm.at[wid], s_v)

        copies = [
            pltpu.async_copy(dist_hbm.at[src_v.at[j]],
                             d_v.at[pl.ds(j * 128, 128)], sem)
            for j in range(KROWS)
        ]
        for cp in copies:
            cp.wait()

        def body(i, carry):
            sl = pl.ds(i * 16, 16)
            w_v[sl] = d_v[sl] * s_v[sl]
            return carry
        lax.fori_loop(0, CH // 16, body, 0)

        for j in range(KROWS):
            pltpu.sync_copy(w_v.at[pl.ds(j * 128, 128)],
                            acc_sh.at[dst_v.at[j]], add=True)

        plsc.subcore_barrier()

        @pl.when(sid == 0)
        def _():
            pltpu.sync_copy(acc_sh, out_hbm.at[cid])

    return sc_scatter


_sc_scatter_cache = []


def _get_sc_scatter():
    if not _sc_scatter_cache:
        _sc_scatter_cache.append(_make_sc_scatter())
    return _sc_scatter_cache[0]


def _final_body(instr_ref, pe_ref, gids_ref, sv_ref, rel_ref, out_ref):
    c = _softmax_cols(pe_ref[...], instr_ref[...])
    c15 = lax.slice(c, (P - 1, 0), (P, 1))
    gids = gids_ref[...]
    g3 = lax.broadcasted_iota(jnp.int32, (G, NROWS, 128), 0)
    oh = gids[None, :, :] == g3
    ohf = oh.astype(jnp.float32)

    def segsm(v):
        mx = jnp.max(jnp.where(oh, v[None, :, :], -1e30),
                     axis=(1, 2), keepdims=True)
        vmax = jnp.sum(ohf * mx, axis=0)
        e = jnp.exp(v - vmax)
        den = jnp.sum(ohf * e[None, :, :], axis=(1, 2), keepdims=True)
        den_n = jnp.sum(ohf * den, axis=0)
        return e / jnp.maximum(den_n, 1e-30)

    rel = rel_ref[0] + rel_ref[1]
    out_ref[...] = c15 * segsm(rel) + (1.0 - c15) * segsm(sv_ref[...])


def _finalize(instr2d, prop_embeds, gids2d, sv2d, rel3d):
    return pl.pallas_call(
        _final_body,
        out_shape=jax.ShapeDtypeStruct((NROWS, 128), jnp.float32),
    )(instr2d, prop_embeds, gids2d, sv2d, rel3d)


def kernel(instruction, prop_embeds, node_attrs, edge_attrs, node_graph_ids,
           edge_indices, Ws_property, W_state, W_relation):
    instr2d = instruction.reshape(1, H)
    wstate_col = W_state.reshape(H, 1)
    wrel_col = W_relation.reshape(H, 1)
    w_t = jnp.swapaxes(Ws_property, 1, 2)
    w_node = w_t[: P - 1]
    w_edge = w_t[P - 1]

    gids_pad = jnp.concatenate(
        [node_graph_ids, jnp.full((NPAD - N,), G, jnp.int32)])
    gids2d = gids_pad.reshape(NROWS, 128)

    state_vals = _node_state_vals(instr2d, prop_embeds, node_attrs,
                                  w_node, wstate_col)
    s = _edge_scalars(instr2d, w_edge, edge_attrs, wrel_col)

    dist2d = _distribution(gids2d)

    src = jnp.concatenate(
        [edge_indices[0], jnp.zeros((EPAD - E,), jnp.int32)]).reshape(
            NTILES, KROWS, 128)
    dst = jnp.concatenate(
        [edge_indices[1], jnp.zeros((EPAD - E,), jnp.int32)]).reshape(
            NTILES, KROWS, 128)
    s_pad = jnp.concatenate(
        [s.reshape(E), jnp.zeros((EPAD - E,), jnp.float32)]).reshape(NTILES, CH)
    rel_parts = _get_sc_scatter()(dist2d.reshape(NPAD), src, dst, s_pad)

    sv2d = jnp.concatenate(
        [state_vals.reshape(N), jnp.zeros((NPAD - N,), jnp.float32)]).reshape(
            NROWS, 128)
    out2d = _finalize(instr2d, prop_embeds, gids2d, sv2d,
                      rel_parts.reshape(2, NROWS, 128))
    return out2d.reshape(NPAD)[:N]


# speedup vs baseline: 5.9934x; 1.0409x over previous
import functools

import jax
import jax.numpy as jnp
from jax import lax
from jax.experimental import pallas as pl
from jax.experimental.pallas import tpu as pltpu
from jax.experimental.pallas import tpu_sc as plsc

G = 64
H = 256
P = 16
N = 10000
E = 160000

NPAD = 10240
EPAD = 163840
NTILES = 32
CH = EPAD // NTILES
KROWS = CH // 128
NROWS = NPAD // 128

BN = 400
BE = 8000


def _softmax_cols(pe, instr):
    pb = pe.astype(jnp.bfloat16).astype(jnp.float32)
    ib = instr.astype(jnp.bfloat16).astype(jnp.float32)
    logits = jnp.sum(pb * ib, axis=1, keepdims=True)
    m = jnp.max(logits, axis=0, keepdims=True)
    ex = jnp.exp(logits - m)
    return ex / jnp.sum(ex, axis=0, keepdims=True)


def _elu(x):
    return jnp.where(x > 0, x, jnp.exp(jnp.minimum(x, 0.0)) - 1.0)


def _node_body(instr_ref, pe_ref, x_ref, w_ref, wstate_ref, out_ref):
    c = _softmax_cols(pe_ref[...], instr_ref[...])
    acc = jnp.zeros((BN, H), jnp.float32)
    for p in range(P - 1):
        cp = lax.slice(c, (p, 0), (p + 1, 1))
        acc = acc + jnp.dot(x_ref[:, p, :], w_ref[p],
                            preferred_element_type=jnp.float32) * cp
    y = _elu(acc * instr_ref[...])
    out_ref[...] = jnp.dot(y, wstate_ref[...],
                           preferred_element_type=jnp.float32)


def _node_state_vals(instr2d, prop_embeds, node_attrs, w_node, wstate_col):
    return pl.pallas_call(
        _node_body,
        grid=(N // BN,),
        in_specs=[
            pl.BlockSpec((1, H), lambda i: (0, 0)),
            pl.BlockSpec((P, H), lambda i: (0, 0)),
            pl.BlockSpec((BN, P - 1, H), lambda i: (i, 0, 0)),
            pl.BlockSpec((P - 1, H, H), lambda i: (0, 0, 0)),
            pl.BlockSpec((H, 1), lambda i: (0, 0)),
        ],
        out_specs=pl.BlockSpec((BN, 1), lambda i: (i, 0)),
        out_shape=jax.ShapeDtypeStruct((N, 1), jnp.float32),
    )(instr2d, prop_embeds, node_attrs, w_node, wstate_col)


def _edge_body(instr_ref, we_ref, x_ref, wrel_ref, out_ref):
    pre = jnp.dot(x_ref[...], we_ref[...],
                  preferred_element_type=jnp.float32) * instr_ref[...]
    y = _elu(pre)
    out_ref[...] = jnp.dot(y, wrel_ref[...],
                           preferred_element_type=jnp.float32)


def _edge_scalars(instr2d, w_edge, edge_attrs, wrel_col):
    return pl.pallas_call(
        _edge_body,
        grid=(E // BE,),
        in_specs=[
            pl.BlockSpec((1, H), lambda i: (0, 0)),
            pl.BlockSpec((H, H), lambda i: (0, 0)),
            pl.BlockSpec((BE, H), lambda i: (i, 0)),
            pl.BlockSpec((H, 1), lambda i: (0, 0)),
        ],
        out_specs=pl.BlockSpec((BE, 1), lambda i: (i, 0)),
        out_shape=jax.ShapeDtypeStruct((E, 1), jnp.float32),
    )(instr2d, w_edge, edge_attrs, wrel_col)


def _dist_body(gids_ref, out_ref):
    gids = gids_ref[...]
    g3 = lax.broadcasted_iota(jnp.int32, (G, NROWS, 128), 0)
    oh = (gids[None, :, :] == g3).astype(jnp.float32)
    cnt = jnp.sum(oh, axis=(1, 2), keepdims=True)
    inv = 1.0 / jnp.maximum(cnt, 1.0)
    out_ref[...] = jnp.sum(oh * inv, axis=0)


def _distribution(gids2d):
    return pl.pallas_call(
        _dist_body,
        out_shape=jax.ShapeDtypeStruct((NROWS, 128), jnp.float32),
    )(gids2d)


def _make_sc_scatter():
    mesh = plsc.VectorSubcoreMesh(core_axis_name="c", subcore_axis_name="s",
                                  num_cores=2, num_subcores=16)

    @functools.partial(
        pl.kernel,
        mesh=mesh,
        out_type=jax.ShapeDtypeStruct((2, NPAD), jnp.float32),
        scratch_types=[
            pltpu.VMEM((CH,), jnp.float32),
            pltpu.VMEM((KROWS, 128), jnp.int32),
            pltpu.VMEM((KROWS, 128), jnp.int32),
            pltpu.VMEM((CH,), jnp.float32),
            pltpu.VMEM((CH,), jnp.float32),
            pltpu.VMEM_SHARED((NPAD,), jnp.float32),
            pltpu.VMEM_SHARED((NPAD,), jnp.float32),
            pltpu.SemaphoreType.DMA,
        ],
    )
    def sc_scatter(dist_hbm, src_hbm, dst_hbm, s_hbm, out_hbm,
                   d_v, src_v, dst_v, s_v, w_v, acc_sh, dist_sh, sem):
        cid = lax.axis_index("c")
        sid = lax.axis_index("s")
        wid = cid * 16 + sid

        stage = [pltpu.async_copy(src_hbm.at[wid], src_v, sem),
                 pltpu.async_copy(dst_hbm.at[wid], dst_v, sem),
                 pltpu.async_copy(s_hbm.at[wid], s_v, sem)]

        @pl.when(sid == 0)
        def _():
            def zb(i, carry):
                w_v[pl.ds(i * 16, 16)] = jnp.zeros((16,), jnp.float32)
                return carry
            lax.fori_loop(0, CH // 16, zb, 0)
            pltpu.sync_copy(dist_hbm, dist_sh)
            for r in range(NPAD // CH):
                pltpu.sync_copy(w_v, acc_sh.at[pl.ds(r * CH, CH)])

        for cp in stage:
            cp.wait()
        plsc.subcore_barrier()

        copies = [
            pltpu.async_copy(dist_sh.at[src_v.at[j]],
                             d_v.at[pl.ds(j * 128, 128)], sem)
            for j in range(KROWS)
        ]
        for cp in copies:
            cp.wait()

        def body(i, carry):
            sl = pl.ds(i * 16, 16)
            w_v[sl] = d_v[sl] * s_v[sl]
            return carry
        lax.fori_loop(0, CH // 16, body, 0)

        adds = [
            pltpu.async_copy(w_v.at[pl.ds(j * 128, 128)],
                             acc_sh.at[dst_v.at[j]], sem, add=True)
            for j in range(KROWS)
        ]
        for cp in adds:
            cp.wait()

        plsc.subcore_barrier()

        @pl.when(sid == 0)
        def _():
            pltpu.sync_copy(acc_sh, out_hbm.at[cid])

    return sc_scatter


_sc_scatter_cache = []


def _get_sc_scatter():
    if not _sc_scatter_cache:
        _sc_scatter_cache.append(_make_sc_scatter())
    return _sc_scatter_cache[0]


def _final_body(instr_ref, pe_ref, gids_ref, sv_ref, rel_ref, out_ref):
    c = _softmax_cols(pe_ref[...], instr_ref[...])
    c15 = lax.slice(c, (P - 1, 0), (P, 1))
    gids = gids_ref[...]
    g3 = lax.broadcasted_iota(jnp.int32, (G, NROWS, 128), 0)
    oh = gids[None, :, :] == g3
    ohf = oh.astype(jnp.float32)

    def segsm(v):
        mx = jnp.max(jnp.where(oh, v[None, :, :], -1e30),
                     axis=(1, 2), keepdims=True)
        vmax = jnp.sum(ohf * mx, axis=0)
        e = jnp.exp(v - vmax)
        den = jnp.sum(ohf * e[None, :, :], axis=(1, 2), keepdims=True)
        den_n = jnp.sum(ohf * den, axis=0)
        return e / jnp.maximum(den_n, 1e-30)

    rel = rel_ref[0] + rel_ref[1]
    out_ref[...] = c15 * segsm(rel) + (1.0 - c15) * segsm(sv_ref[...])


def _finalize(instr2d, prop_embeds, gids2d, sv2d, rel3d):
    return pl.pallas_call(
        _final_body,
        out_shape=jax.ShapeDtypeStruct((NROWS, 128), jnp.float32),
    )(instr2d, prop_embeds, gids2d, sv2d, rel3d)


def kernel(instruction, prop_embeds, node_attrs, edge_attrs, node_graph_ids,
           edge_indices, Ws_property, W_state, W_relation):
    instr2d = instruction.reshape(1, H)
    wstate_col = W_state.reshape(H, 1)
    wrel_col = W_relation.reshape(H, 1)
    w_t = jnp.swapaxes(Ws_property, 1, 2)
    w_node = w_t[: P - 1]
    w_edge = w_t[P - 1]

    gids_pad = jnp.concatenate(
        [node_graph_ids, jnp.full((NPAD - N,), G, jnp.int32)])
    gids2d = gids_pad.reshape(NROWS, 128)

    state_vals = _node_state_vals(instr2d, prop_embeds, node_attrs,
                                  w_node, wstate_col)
    s = _edge_scalars(instr2d, w_edge, edge_attrs, wrel_col)

    dist2d = _distribution(gids2d)

    src = jnp.concatenate(
        [edge_indices[0], jnp.zeros((EPAD - E,), jnp.int32)]).reshape(
            NTILES, KROWS, 128)
    dst = jnp.concatenate(
        [edge_indices[1], jnp.zeros((EPAD - E,), jnp.int32)]).reshape(
            NTILES, KROWS, 128)
    s_pad = jnp.concatenate(
        [s.reshape(E), jnp.zeros((EPAD - E,), jnp.float32)]).reshape(NTILES, CH)
    rel_parts = _get_sc_scatter()(dist2d.reshape(NPAD), src, dst, s_pad)

    sv2d = jnp.concatenate(
        [state_vals.reshape(N), jnp.zeros((NPAD - N,), jnp.float32)]).reshape(
            NROWS, 128)
    out2d = _finalize(instr2d, prop_embeds, gids2d, sv2d,
                      rel_parts.reshape(2, NROWS, 128))
    return out2d.reshape(NPAD)[:N]


# speedup vs baseline: 6.2542x; 1.0435x over previous
import functools

import jax
import jax.numpy as jnp
from jax import lax
from jax.experimental import pallas as pl
from jax.experimental.pallas import tpu as pltpu
from jax.experimental.pallas import tpu_sc as plsc

G = 64
H = 256
P = 16
N = 10000
E = 160000

NPAD = 10240
EPAD = 163840
NTILES = 32
CH = EPAD // NTILES
KROWS = CH // 128
NROWS = NPAD // 128

BN = 1000
BE = 16000


def _softmax_cols(pe, instr):
    pb = pe.astype(jnp.bfloat16).astype(jnp.float32)
    ib = instr.astype(jnp.bfloat16).astype(jnp.float32)
    logits = jnp.sum(pb * ib, axis=1, keepdims=True)
    m = jnp.max(logits, axis=0, keepdims=True)
    ex = jnp.exp(logits - m)
    return ex / jnp.sum(ex, axis=0, keepdims=True)


def _elu(x):
    return jnp.where(x > 0, x, jnp.exp(jnp.minimum(x, 0.0)) - 1.0)


def _node_body(instr_ref, pe_ref, x_ref, w_ref, wstate_ref, out_ref):
    c = _softmax_cols(pe_ref[...], instr_ref[...])
    acc = jnp.zeros((BN, H), jnp.float32)
    for p in range(P - 1):
        cp = lax.slice(c, (p, 0), (p + 1, 1))
        acc = acc + jnp.dot(x_ref[:, p, :], w_ref[p],
                            preferred_element_type=jnp.float32) * cp
    y = _elu(acc * instr_ref[...])
    out_ref[...] = jnp.dot(y, wstate_ref[...],
                           preferred_element_type=jnp.float32)


def _node_state_vals(instr2d, prop_embeds, node_attrs, w_node, wstate_col):
    return pl.pallas_call(
        _node_body,
        grid=(N // BN,),
        in_specs=[
            pl.BlockSpec((1, H), lambda i: (0, 0)),
            pl.BlockSpec((P, H), lambda i: (0, 0)),
            pl.BlockSpec((BN, P - 1, H), lambda i: (i, 0, 0)),
            pl.BlockSpec((P - 1, H, H), lambda i: (0, 0, 0)),
            pl.BlockSpec((H, 1), lambda i: (0, 0)),
        ],
        out_specs=pl.BlockSpec((BN, 1), lambda i: (i, 0)),
        out_shape=jax.ShapeDtypeStruct((N, 1), jnp.float32),
    )(instr2d, prop_embeds, node_attrs, w_node, wstate_col)


def _edge_body(instr_ref, we_ref, x_ref, wrel_ref, out_ref):
    pre = jnp.dot(x_ref[...], we_ref[...],
                  preferred_element_type=jnp.float32) * instr_ref[...]
    y = _elu(pre)
    out_ref[...] = jnp.dot(y, wrel_ref[...],
                           preferred_element_type=jnp.float32)


def _edge_scalars(instr2d, w_edge, edge_attrs, wrel_col):
    return pl.pallas_call(
        _edge_body,
        grid=(E // BE,),
        in_specs=[
            pl.BlockSpec((1, H), lambda i: (0, 0)),
            pl.BlockSpec((H, H), lambda i: (0, 0)),
            pl.BlockSpec((BE, H), lambda i: (i, 0)),
            pl.BlockSpec((H, 1), lambda i: (0, 0)),
        ],
        out_specs=pl.BlockSpec((BE, 1), lambda i: (i, 0)),
        out_shape=jax.ShapeDtypeStruct((E, 1), jnp.float32),
    )(instr2d, w_edge, edge_attrs, wrel_col)


def _dist_body(gids_ref, out_ref):
    gids = gids_ref[...]
    g3 = lax.broadcasted_iota(jnp.int32, (G, NROWS, 128), 0)
    oh = (gids[None, :, :] == g3).astype(jnp.float32)
    cnt = jnp.sum(oh, axis=(1, 2), keepdims=True)
    inv = 1.0 / jnp.maximum(cnt, 1.0)
    out_ref[...] = jnp.sum(oh * inv, axis=0)


def _distribution(gids2d):
    return pl.pallas_call(
        _dist_body,
        out_shape=jax.ShapeDtypeStruct((NROWS, 128), jnp.float32),
    )(gids2d)


def _make_sc_scatter():
    mesh = plsc.VectorSubcoreMesh(core_axis_name="c", subcore_axis_name="s",
                                  num_cores=2, num_subcores=16)

    @functools.partial(
        pl.kernel,
        mesh=mesh,
        out_type=jax.ShapeDtypeStruct((2, NPAD), jnp.float32),
        scratch_types=[
            pltpu.VMEM((CH,), jnp.float32),
            pltpu.VMEM((KROWS, 128), jnp.int32),
            pltpu.VMEM((KROWS, 128), jnp.int32),
            pltpu.VMEM((CH,), jnp.float32),
            pltpu.VMEM((CH,), jnp.float32),
            pltpu.VMEM_SHARED((NPAD,), jnp.float32),
            pltpu.VMEM_SHARED((NPAD,), jnp.float32),
            pltpu.SemaphoreType.DMA,
        ],
    )
    def sc_scatter(dist_hbm, src_hbm, dst_hbm, s_hbm, out_hbm,
                   d_v, src_v, dst_v, s_v, w_v, acc_sh, dist_sh, sem):
        cid = lax.axis_index("c")
        sid = lax.axis_index("s")
        wid = cid * 16 + sid

        stage = [pltpu.async_copy(src_hbm.at[wid], src_v, sem),
                 pltpu.async_copy(dst_hbm.at[wid], dst_v, sem),
                 pltpu.async_copy(s_hbm.at[wid], s_v, sem)]

        @pl.when(sid == 0)
        def _():
            def zb(i, carry):
                w_v[pl.ds(i * 16, 16)] = jnp.zeros((16,), jnp.float32)
                return carry
            lax.fori_loop(0, CH // 16, zb, 0)
            pltpu.sync_copy(dist_hbm, dist_sh)
            for r in range(NPAD // CH):
                pltpu.sync_copy(w_v, acc_sh.at[pl.ds(r * CH, CH)])

        for cp in stage:
            cp.wait()
        plsc.subcore_barrier()

        copies = [
            pltpu.async_copy(dist_sh.at[src_v.at[j]],
                             d_v.at[pl.ds(j * 128, 128)], sem)
            for j in range(KROWS)
        ]
        for cp in copies:
            cp.wait()

        def body(i, carry):
            sl = pl.ds(i * 16, 16)
            w_v[sl] = d_v[sl] * s_v[sl]
            return carry
        lax.fori_loop(0, CH // 16, body, 0)

        adds = [
            pltpu.async_copy(w_v.at[pl.ds(j * 128, 128)],
                             acc_sh.at[dst_v.at[j]], sem, add=True)
            for j in range(KROWS)
        ]
        for cp in adds:
            cp.wait()

        plsc.subcore_barrier()

        @pl.when(sid == 0)
        def _():
            pltpu.sync_copy(acc_sh, out_hbm.at[cid])

    return sc_scatter


_sc_scatter_cache = []


def _get_sc_scatter():
    if not _sc_scatter_cache:
        _sc_scatter_cache.append(_make_sc_scatter())
    return _sc_scatter_cache[0]


def _final_body(instr_ref, pe_ref, gids_ref, sv_ref, rel_ref, out_ref):
    c = _softmax_cols(pe_ref[...], instr_ref[...])
    c15 = lax.slice(c, (P - 1, 0), (P, 1))
    gids = gids_ref[...]
    g3 = lax.broadcasted_iota(jnp.int32, (G, NROWS, 128), 0)
    oh = gids[None, :, :] == g3
    ohf = oh.astype(jnp.float32)

    def segsm(v):
        mx = jnp.max(jnp.where(oh, v[None, :, :], -1e30),
                     axis=(1, 2), keepdims=True)
        vmax = jnp.sum(ohf * mx, axis=0)
        e = jnp.exp(v - vmax)
        den = jnp.sum(ohf * e[None, :, :], axis=(1, 2), keepdims=True)
        den_n = jnp.sum(ohf * den, axis=0)
        return e / jnp.maximum(den_n, 1e-30)

    rel = rel_ref[0] + rel_ref[1]
    out_ref[...] = c15 * segsm(rel) + (1.0 - c15) * segsm(sv_ref[...])


def _finalize(instr2d, prop_embeds, gids2d, sv2d, rel3d):
    return pl.pallas_call(
        _final_body,
        out_shape=jax.ShapeDtypeStruct((NROWS, 128), jnp.float32),
    )(instr2d, prop_embeds, gids2d, sv2d, rel3d)


def kernel(instruction, prop_embeds, node_attrs, edge_attrs, node_graph_ids,
           edge_indices, Ws_property, W_state, W_relation):
    instr2d = instruction.reshape(1, H)
    wstate_col = W_state.reshape(H, 1)
    wrel_col = W_relation.reshape(H, 1)
    w_t = jnp.swapaxes(Ws_property, 1, 2)
    w_node = w_t[: P - 1]
    w_edge = w_t[P - 1]

    gids_pad = jnp.concatenate(
        [node_graph_ids, jnp.full((NPAD - N,), G, jnp.int32)])
    gids2d = gids_pad.reshape(NROWS, 128)

    s = _edge_scalars(instr2d, w_edge, edge_attrs, wrel_col)
    dist2d = _distribution(gids2d)

    src = jnp.concatenate(
        [edge_indices[0], jnp.zeros((EPAD - E,), jnp.int32)]).reshape(
            NTILES, KROWS, 128)
    dst = jnp.concatenate(
        [edge_indices[1], jnp.zeros((EPAD - E,), jnp.int32)]).reshape(
            NTILES, KROWS, 128)
    s_pad = jnp.concatenate(
        [s.reshape(E), jnp.zeros((EPAD - E,), jnp.float32)]).reshape(NTILES, CH)
    rel_parts = _get_sc_scatter()(dist2d.reshape(NPAD), src, dst, s_pad)

    state_vals = _node_state_vals(instr2d, prop_embeds, node_attrs,
                                  w_node, wstate_col)

    sv2d = jnp.concatenate(
        [state_vals.reshape(N), jnp.zeros((NPAD - N,), jnp.float32)]).reshape(
            NROWS, 128)
    out2d = _finalize(instr2d, prop_embeds, gids2d, sv2d,
                      rel_parts.reshape(2, NROWS, 128))
    return out2d.reshape(NPAD)[:N]


# speedup vs baseline: 6.2730x; 1.0030x over previous
import functools

import jax
import jax.numpy as jnp
from jax import lax
from jax.experimental import pallas as pl
from jax.experimental.pallas import tpu as pltpu
from jax.experimental.pallas import tpu_sc as plsc

G = 64
H = 256
P = 16
N = 10000
E = 160000

NPAD = 10240
EPAD = 163840
NTILES = 32
CH = EPAD // NTILES
KROWS = CH // 128
NROWS = NPAD // 128

BN = 1000
BE = 16000


def _softmax_cols(pe, instr):
    pb = pe.astype(jnp.bfloat16).astype(jnp.float32)
    ib = instr.astype(jnp.bfloat16).astype(jnp.float32)
    logits = jnp.sum(pb * ib, axis=1, keepdims=True)
    m = jnp.max(logits, axis=0, keepdims=True)
    ex = jnp.exp(logits - m)
    return ex / jnp.sum(ex, axis=0, keepdims=True)


def _elu(x):
    return jnp.where(x > 0, x, jnp.exp(jnp.minimum(x, 0.0)) - 1.0)


def _node_body(instr_ref, pe_ref, x_ref, w_ref, wstate_ref, out_ref):
    c = _softmax_cols(pe_ref[...], instr_ref[...])
    acc = jnp.zeros((BN, H), jnp.float32)
    for p in range(P - 1):
        cp = lax.slice(c, (p, 0), (p + 1, 1))
        acc = acc + jnp.dot(x_ref[:, p, :], w_ref[p],
                            preferred_element_type=jnp.float32) * cp
    y = _elu(acc * instr_ref[...])
    out_ref[...] = jnp.dot(y, wstate_ref[...],
                           preferred_element_type=jnp.float32)


def _node_state_vals(instr2d, prop_embeds, node_attrs, w_node, wstate_col):
    return pl.pallas_call(
        _node_body,
        grid=(N // BN,),
        in_specs=[
            pl.BlockSpec((1, H), lambda i: (0, 0)),
            pl.BlockSpec((P, H), lambda i: (0, 0)),
            pl.BlockSpec((BN, P - 1, H), lambda i: (i, 0, 0)),
            pl.BlockSpec((P - 1, H, H), lambda i: (0, 0, 0)),
            pl.BlockSpec((H, 1), lambda i: (0, 0)),
        ],
        out_specs=pl.BlockSpec((BN, 1), lambda i: (i, 0)),
        out_shape=jax.ShapeDtypeStruct((N, 1), jnp.float32),
    )(instr2d, prop_embeds, node_attrs, w_node, wstate_col)


def _edge_body(instr_ref, we_ref, x_ref, wrel_ref, out_ref):
    pre = jnp.dot(x_ref[...], we_ref[...],
                  preferred_element_type=jnp.float32) * instr_ref[...]
    y = _elu(pre)
    out_ref[...] = jnp.dot(y, wrel_ref[...],
                           preferred_element_type=jnp.float32)


def _edge_scalars(instr2d, w_edge, edge_attrs, wrel_col):
    return pl.pallas_call(
        _edge_body,
        grid=(E // BE,),
        in_specs=[
            pl.BlockSpec((1, H), lambda i: (0, 0)),
            pl.BlockSpec((H, H), lambda i: (0, 0)),
            pl.BlockSpec((BE, H), lambda i: (i, 0)),
            pl.BlockSpec((H, 1), lambda i: (0, 0)),
        ],
        out_specs=pl.BlockSpec((BE, 1), lambda i: (i, 0)),
        out_shape=jax.ShapeDtypeStruct((E, 1), jnp.float32),
    )(instr2d, w_edge, edge_attrs, wrel_col)


def _dist_body(gids_ref, out_ref):
    gids = gids_ref[...]
    g3 = lax.broadcasted_iota(jnp.int32, (G, NROWS, 128), 0)
    oh = (gids[None, :, :] == g3).astype(jnp.float32)
    cnt = jnp.sum(oh, axis=(1, 2), keepdims=True)
    inv = 1.0 / jnp.maximum(cnt, 1.0)
    out_ref[...] = jnp.sum(oh * inv, axis=0)


def _distribution(gids2d):
    return pl.pallas_call(
        _dist_body,
        out_shape=jax.ShapeDtypeStruct((NROWS, 128), jnp.float32),
    )(gids2d)


def _make_sc_scatter():
    mesh = plsc.VectorSubcoreMesh(core_axis_name="c", subcore_axis_name="s",
                                  num_cores=2, num_subcores=16)

    @functools.partial(
        pl.kernel,
        mesh=mesh,
        out_type=jax.ShapeDtypeStruct((2, NPAD), jnp.float32),
        scratch_types=[
            pltpu.VMEM((CH,), jnp.float32),
            pltpu.VMEM((KROWS, 128), jnp.int32),
            pltpu.VMEM((KROWS, 128), jnp.int32),
            pltpu.VMEM((CH,), jnp.float32),
            pltpu.VMEM((CH,), jnp.float32),
            pltpu.VMEM_SHARED((NPAD,), jnp.float32),
            pltpu.VMEM_SHARED((NPAD,), jnp.float32),
            pltpu.SemaphoreType.DMA,
        ],
    )
    def sc_scatter(dist_hbm, src_hbm, dst_hbm, s_hbm, out_hbm,
                   d_v, src_v, dst_v, s_v, w_v, acc_sh, dist_sh, sem):
        cid = lax.axis_index("c")
        sid = lax.axis_index("s")
        wid = cid * 16 + sid

        stage = [pltpu.async_copy(src_hbm.at[wid], src_v, sem),
                 pltpu.async_copy(dst_hbm.at[wid], dst_v, sem),
                 pltpu.async_copy(s_hbm.at[wid], s_v, sem)]

        @pl.when(sid == 0)
        def _():
            def zb(i, carry):
                w_v[pl.ds(i * 16, 16)] = jnp.zeros((16,), jnp.float32)
                return carry
            lax.fori_loop(0, CH // 16, zb, 0)
            pltpu.sync_copy(dist_hbm, dist_sh)
            for r in range(NPAD // CH):
                pltpu.sync_copy(w_v, acc_sh.at[pl.ds(r * CH, CH)])

        for cp in stage:
            cp.wait()
        plsc.subcore_barrier()

        copies = [
            pltpu.async_copy(dist_sh.at[src_v.at[j]],
                             d_v.at[pl.ds(j * 128, 128)], sem)
            for j in range(KROWS)
        ]
        for cp in copies:
            cp.wait()

        def body(i, carry):
            sl = pl.ds(i * 16, 16)
            w_v[sl] = d_v[sl] * s_v[sl]
            return carry
        lax.fori_loop(0, CH // 16, body, 0)

        adds = [
            pltpu.async_copy(w_v.at[pl.ds(j * 128, 128)],
                             acc_sh.at[dst_v.at[j]], sem, add=True)
            for j in range(KROWS)
        ]
        for cp in adds:
            cp.wait()

        plsc.subcore_barrier()

        @pl.when(sid == 0)
        def _():
            pltpu.sync_copy(acc_sh, out_hbm.at[cid])

    return sc_scatter


_sc_scatter_cache = []


def _get_sc_scatter():
    if not _sc_scatter_cache:
        _sc_scatter_cache.append(_make_sc_scatter())
    return _sc_scatter_cache[0]


def _final_body(instr_ref, pe_ref, gids_ref, sv_ref, rel_ref, out_ref):
    c = _softmax_cols(pe_ref[...], instr_ref[...])
    c15 = lax.slice(c, (P - 1, 0), (P, 1))
    gids = gids_ref[...]
    g3 = lax.broadcasted_iota(jnp.int32, (G, NROWS, 128), 0)
    oh = gids[None, :, :] == g3
    ohf = oh.astype(jnp.float32)

    def segsm(v):
        mx = jnp.max(jnp.where(oh, v[None, :, :], -1e30),
                     axis=(1, 2), keepdims=True)
        vmax = jnp.sum(ohf * mx, axis=0)
        e = jnp.exp(v - vmax)
        den = jnp.sum(ohf * e[None, :, :], axis=(1, 2), keepdims=True)
        den_n = jnp.sum(ohf * den, axis=0)
        return e / jnp.maximum(den_n, 1e-30)

    rel = rel_ref[0] + rel_ref[1]
    out_ref[...] = c15 * segsm(rel) + (1.0 - c15) * segsm(sv_ref[...])


def _finalize(instr2d, prop_embeds, gids2d, sv2d, rel3d):
    return pl.pallas_call(
        _final_body,
        out_shape=jax.ShapeDtypeStruct((NROWS, 128), jnp.float32),
    )(instr2d, prop_embeds, gids2d, sv2d, rel3d)


def kernel(instruction, prop_embeds, node_attrs, edge_attrs, node_graph_ids,
           edge_indices, Ws_property, W_state, W_relation):
    instr2d = instruction.reshape(1, H)
    wstate_col = W_state.reshape(H, 1)
    wrel_col = W_relation.reshape(H, 1)
    w_t = jnp.swapaxes(Ws_property, 1, 2)
    w_node = w_t[: P - 1]
    w_edge = w_t[P - 1]

    gids_pad = jnp.concatenate(
        [node_graph_ids, jnp.full((NPAD - N,), G, jnp.int32)])
    gids2d = gids_pad.reshape(NROWS, 128)

    s = _edge_scalars(instr2d, w_edge, edge_attrs, wrel_col)
    dist2d = _distribution(gids2d)

    src = jnp.concatenate(
        [edge_indices[0], jnp.zeros((EPAD - E,), jnp.int32)]).reshape(
            NTILES, KROWS, 128)
    dst = jnp.concatenate(
        [edge_indices[1], jnp.zeros((EPAD - E,), jnp.int32)]).reshape(
            NTILES, KROWS, 128)
    s_pad = jnp.concatenate(
        [s.reshape(E), jnp.zeros((EPAD - E,), jnp.float32)]).reshape(NTILES, CH)
    state_vals = _node_state_vals(instr2d, prop_embeds, node_attrs,
                                  w_node, wstate_col)
    rel_parts = _get_sc_scatter()(dist2d.reshape(NPAD), src, dst, s_pad)

    sv2d = jnp.concatenate(
        [state_vals.reshape(N), jnp.zeros((NPAD - N,), jnp.float32)]).reshape(
            NROWS, 128)
    out2d = _finalize(instr2d, prop_embeds, gids2d, sv2d,
                      rel_parts.reshape(2, NROWS, 128))
    return out2d.reshape(NPAD)[:N]
